# baseline (speedup 1.0000x reference)
import functools

import jax
import jax.numpy as jnp
from jax import lax
from jax.experimental import pallas as pl
from jax.experimental.pallas import tpu as pltpu
from jax.experimental.pallas import tpu_sc as plsc

N = 10000
D = 128
E = 320000
NC = 2
NS = 16
NW = NC * NS
CHUNK = 128
RPW = 80
EP = NW * RPW * CHUNK
ROWS = EP // CHUNK
ACC_ROWS = 10240
CNT_ROWS = 10240
CNT_PW = CNT_ROWS // NS
APW = ACC_ROWS // NS


def _agg_body(x_hbm, idx_hbm, sum_out, cnt_out,
              ibuf, rows0, rows1, ones_b, cnt_buf,
              acc, cnts,
              isem0, isem1, isem2, isem3, gsem0, gsem1,
              ssem0, ssem1, csem0, csem1):
    c = lax.axis_index("c")
    s = lax.axis_index("s")
    wid = s * NC + c
    r0 = wid * RPW
    isems = (isem0, isem1, isem2, isem3)
    gbufs = ((rows0, gsem0), (rows1, gsem1))
    ssems = (ssem0, ssem1)
    csems = (csem0, csem1)

    z16 = jnp.zeros((16,), jnp.float32)
    o16 = jnp.ones((16,), jnp.float32)
    for k in range(8):
        ones_b[pl.ds(k * 16, 16)] = o16

    def zrow(r, _):
        for k in range(8):
            rows0[r, pl.ds(k * 16, 16)] = z16
        return _
    lax.fori_loop(0, CHUNK, zrow, None)

    def zcnt(i, _):
        cnt_buf[pl.ds(i * 16, 16)] = z16
        return _
    lax.fori_loop(0, CNT_PW // 16, zcnt, None)

    base = s * APW
    for k in range(APW // CHUNK):
        pltpu.sync_copy(rows0, acc.at[pl.ds(base + k * CHUNK, CHUNK)])
    pltpu.sync_copy(cnt_buf, cnts.at[pl.ds(s * CNT_PW, CNT_PW)])
    plsc.subcore_barrier()

    def idesc(r, b):
        return pltpu.make_async_copy(idx_hbm.at[r0 + r], ibuf.at[b], isems[b])

    def gdesc(b, gb):
        rows, sem = gbufs[gb]
        return pltpu.make_async_copy(x_hbm.at[ibuf.at[b, 0]], rows, sem)

    def sdesc(b, gb):
        rows = gbufs[gb][0]
        return pltpu.make_async_copy(rows, acc.at[ibuf.at[b, 1]], ssems[gb])

    def cdesc(b, gb):
        return pltpu.make_async_copy(ones_b, cnts.at[ibuf.at[b, 1]], csems[gb])

    idesc(0, 0).start()
    idesc(1, 1).start()
    idesc(0, 0).wait()
    gdesc(0, 0).start()

    def outer(m, _):
        for b in range(4):
            r = m * 4 + b
            bp1, bm1 = (b + 1) % 4, (b - 1) % 4
            gb, gbn = b % 2, (b + 1) % 2

            @pl.when(r + 2 < RPW)
            def _pf_idx():
                idesc(r + 2, (b + 2) % 4).start()

            gdesc(b, gb).wait()

            @pl.when(r >= 1)
            def _drain_prev():
                sdesc(bm1, gbn).wait()
                cdesc(bm1, gbn).wait()

            @pl.when(r + 1 < RPW)
            def _next_gather():
                idesc(r + 1, bp1).wait()
                gdesc(bp1, gbn).start()

            rows = gbufs[gb][0]
            didx = ibuf.at[b, 1]
            pltpu.async_copy(rows, acc.at[didx], ssems[gb], add=True)
            pltpu.async_copy(ones_b, cnts.at[didx], csems[gb], add=True)
        return _
    lax.fori_loop(0, RPW // 4, outer, None)

    sdesc(3, 1).wait()
    cdesc(3, 1).wait()

    plsc.subcore_barrier()

    for k in range(APW // CHUNK):
        off = base + k * CHUNK
        pltpu.sync_copy(acc.at[pl.ds(off, CHUNK)], rows0)
        pltpu.sync_copy(rows0, sum_out.at[c].at[pl.ds(off, CHUNK)])
    pltpu.sync_copy(cnts.at[pl.ds(s * CNT_PW, CNT_PW)], cnt_buf)
    pltpu.sync_copy(cnt_buf, cnt_out.at[c].at[s])


@jax.jit
def _aggregate(x, idx2):
    mesh = plsc.VectorSubcoreMesh(core_axis_name="c", subcore_axis_name="s")
    f = pl.kernel(
        _agg_body,
        out_type=[
            jax.ShapeDtypeStruct((NC, ACC_ROWS, D), jnp.float32),
            jax.ShapeDtypeStruct((NC, NS, CNT_PW), jnp.float32),
        ],
        mesh=mesh,
        scratch_types=[
            pltpu.VMEM((4, 2, CHUNK), jnp.int32),
            pltpu.VMEM((CHUNK, D), jnp.float32),
            pltpu.VMEM((CHUNK, D), jnp.float32),
            pltpu.VMEM((CHUNK,), jnp.float32),
            pltpu.VMEM((CNT_PW,), jnp.float32),
            pltpu.VMEM_SHARED((ACC_ROWS, D), jnp.float32),
            pltpu.VMEM_SHARED((CNT_ROWS,), jnp.float32),
        ] + [pltpu.SemaphoreType.DMA] * 10,
    )
    return f(x, idx2)


def _tc_body(sum_ref, cnt_ref, x_ref, wlt_ref, wrt_ref, b_ref, o_ref):
    total = sum_ref[0] + sum_ref[1]
    cnt = cnt_ref[0] + cnt_ref[1]
    mean = total * (1.0 / jnp.maximum(cnt, 1.0))
    o_ref[...] = (
        jnp.dot(mean, wlt_ref[...], preferred_element_type=jnp.float32)
        + jnp.dot(x_ref[...], wrt_ref[...], preferred_element_type=jnp.float32)
        + b_ref[...]
    )


@jax.jit
def _combine(summed, cnt, x, wlt, wrt, b):
    blk = 1000
    grid = N // blk
    return pl.pallas_call(
        _tc_body,
        grid=(grid,),
        in_specs=[
            pl.BlockSpec((NC, blk, D), lambda i: (0, i, 0)),
            pl.BlockSpec((NC, blk, 1), lambda i: (0, i, 0)),
            pl.BlockSpec((blk, D), lambda i: (i, 0)),
            pl.BlockSpec((D, D), lambda i: (0, 0)),
            pl.BlockSpec((D, D), lambda i: (0, 0)),
            pl.BlockSpec((1, D), lambda i: (0, 0)),
        ],
        out_specs=pl.BlockSpec((blk, D), lambda i: (i, 0)),
        out_shape=jax.ShapeDtypeStruct((N, D), jnp.float32),
    )(summed, cnt, x, wlt, wrt, b)


def kernel(x, edge_index, W_l, b_l, W_r):
    src = edge_index[0].astype(jnp.int32)
    dst = edge_index[1].astype(jnp.int32)
    pad = EP - E
    src2 = jnp.concatenate([src, jnp.zeros((pad,), jnp.int32)]).reshape(ROWS, CHUNK)
    dst2 = jnp.concatenate([dst, jnp.full((pad,), N, jnp.int32)]).reshape(ROWS, CHUNK)
    core_of_row = (jnp.arange(ROWS, dtype=jnp.int32) // RPW) % NC
    src2 = src2 + core_of_row[:, None] * N
    idx2 = jnp.stack([src2, dst2], axis=1)
    xcat = jnp.concatenate([x, x], axis=0)
    summed, cnts = _aggregate(xcat, idx2)
    cnt = cnts.reshape(NC, CNT_ROWS, 1)
    return _combine(summed, cnt, x, W_l.T, W_r.T, b_l.reshape(1, D))


# speedup vs baseline: 1.2420x; 1.2420x over previous
import functools

import jax
import jax.numpy as jnp
from jax import lax
from jax.experimental import pallas as pl
from jax.experimental.pallas import tpu as pltpu
from jax.experimental.pallas import tpu_sc as plsc

N = 10000
D = 128
E = 320000
NC = 2
NS = 16
NW = NC * NS
DA = 144
CHUNK = 128
RPW = 80
EP = NW * RPW * CHUNK
ROWS = EP // CHUNK
ACC_ROWS = 10240
APW = ACC_ROWS // NS


def _agg_body(x_hbm, idx_hbm, sum_out,
              ibuf, rows0, rows1,
              acc,
              isem0, isem1, isem2, isem3, gsem0, gsem1):
    c = lax.axis_index("c")
    s = lax.axis_index("s")
    wid = s * NC + c
    r0 = wid * RPW
    isems = (isem0, isem1, isem2, isem3)
    gbufs = ((rows0, gsem0), (rows1, gsem1))

    z16 = jnp.zeros((16,), jnp.float32)

    def zrow(r, _):
        for k in range(DA // 16):
            rows0[r, pl.ds(k * 16, 16)] = z16
        return _
    lax.fori_loop(0, CHUNK, zrow, None)

    base = s * APW
    for k in range(APW // CHUNK):
        pltpu.sync_copy(rows0, acc.at[pl.ds(base + k * CHUNK, CHUNK)])
    plsc.subcore_barrier()

    def idesc(r, b):
        return pltpu.make_async_copy(idx_hbm.at[r0 + r], ibuf.at[b], isems[b])

    def gdesc(b, gb):
        rows, sem = gbufs[gb]
        return pltpu.make_async_copy(x_hbm.at[ibuf.at[b, 0]], rows, sem)

    idesc(0, 0).start()
    idesc(1, 1).start()
    idesc(0, 0).wait()
    gdesc(0, 0).start()

    def outer(m, _):
        for b in range(4):
            r = m * 4 + b
            bp1 = (b + 1) % 4
            gb, gbn = b % 2, (b + 1) % 2

            @pl.when(r + 2 < RPW)
            def _pf_idx():
                idesc(r + 2, (b + 2) % 4).start()

            @pl.when(r + 1 < RPW)
            def _next_gather():
                idesc(r + 1, bp1).wait()
                gdesc(bp1, gbn).start()

            gdesc(b, gb).wait()
            rows = gbufs[gb][0]
            pltpu.sync_copy(rows, acc.at[ibuf.at[b, 1]], add=True)
        return _
    lax.fori_loop(0, RPW // 4, outer, None)

    plsc.subcore_barrier()

    for k in range(APW // CHUNK):
        off = base + k * CHUNK
        pltpu.sync_copy(acc.at[pl.ds(off, CHUNK)], rows0)
        pltpu.sync_copy(rows0, sum_out.at[c].at[pl.ds(off, CHUNK)])


@jax.jit
def _aggregate(xa, idx2):
    mesh = plsc.VectorSubcoreMesh(core_axis_name="c", subcore_axis_name="s")
    f = pl.kernel(
        _agg_body,
        out_type=jax.ShapeDtypeStruct((NC, ACC_ROWS, DA), jnp.float32),
        mesh=mesh,
        scratch_types=[
            pltpu.VMEM((4, 2, CHUNK), jnp.int32),
            pltpu.VMEM((CHUNK, DA), jnp.float32),
            pltpu.VMEM((CHUNK, DA), jnp.float32),
            pltpu.VMEM_SHARED((ACC_ROWS, DA), jnp.float32),
        ] + [pltpu.SemaphoreType.DMA] * 6,
        compiler_params=pltpu.CompilerParams(use_tc_tiling_on_sc=False),
    )
    return f(xa, idx2)


def _tc_body(sum_ref, x_ref, wlt_ref, wrt_ref, b_ref, o_ref):
    total = sum_ref[0] + sum_ref[1]
    cnt = total[:, D:D + 1]
    mean = total[:, :D] * (1.0 / jnp.maximum(cnt, 1.0))
    o_ref[...] = (
        jnp.dot(mean, wlt_ref[...], preferred_element_type=jnp.float32)
        + jnp.dot(x_ref[...], wrt_ref[...], preferred_element_type=jnp.float32)
        + b_ref[...]
    )


@jax.jit
def _combine(summed, x, wlt, wrt, b):
    blk = 1000
    grid = N // blk
    return pl.pallas_call(
        _tc_body,
        grid=(grid,),
        in_specs=[
            pl.BlockSpec((NC, blk, DA), lambda i: (0, i, 0)),
            pl.BlockSpec((blk, D), lambda i: (i, 0)),
            pl.BlockSpec((D, D), lambda i: (0, 0)),
            pl.BlockSpec((D, D), lambda i: (0, 0)),
            pl.BlockSpec((1, D), lambda i: (0, 0)),
        ],
        out_specs=pl.BlockSpec((blk, D), lambda i: (i, 0)),
        out_shape=jax.ShapeDtypeStruct((N, D), jnp.float32),
    )(summed, x, wlt, wrt, b)


def kernel(x, edge_index, W_l, b_l, W_r):
    src = edge_index[0].astype(jnp.int32)
    dst = edge_index[1].astype(jnp.int32)
    pad = EP - E
    src2 = jnp.concatenate([src, jnp.zeros((pad,), jnp.int32)]).reshape(ROWS, CHUNK)
    dst2 = jnp.concatenate([dst, jnp.full((pad,), N, jnp.int32)]).reshape(ROWS, CHUNK)
    idx2 = jnp.stack([src2, dst2], axis=1)
    xa = jnp.concatenate(
        [x, jnp.ones((N, 1), jnp.float32), jnp.zeros((N, DA - D - 1), jnp.float32)],
        axis=1,
    )
    summed = _aggregate(xa, idx2)
    return _combine(summed, x, W_l.T, W_r.T, b_l.reshape(1, D))


# speedup vs baseline: 2.0780x; 1.6731x over previous
import functools

import jax
import jax.numpy as jnp
from jax import lax
from jax.experimental import pallas as pl
from jax.experimental.pallas import tpu as pltpu
from jax.experimental.pallas import tpu_sc as plsc

N = 10000
D = 128
E = 320000
NC = 2
NS = 16
NW = NC * NS
DH = 64
NP = 2
CHUNK = 128
RPW = 80
EP = NW * RPW * CHUNK
ROWS = EP // CHUNK
ACC_ROWS = 10240
CNT_ROWS = 10240
CNT_PW = CNT_ROWS // NS
APW = ACC_ROWS // NS
XPW = N // NS


def _agg_body(x0_hbm, x1_hbm, idx_hbm, sum_out, cnt_out,
              ibuf, rows0, rows1, ones_b, cnt_buf,
              xs, acc, cnts,
              isem0, isem1, isem2, isem3, gsem0, gsem1):
    c = lax.axis_index("c")
    s = lax.axis_index("s")
    wid = s * NC + c
    r0 = wid * RPW
    isems = (isem0, isem1, isem2, isem3)
    gbufs = ((rows0, gsem0), (rows1, gsem1))
    base = s * APW
    xbase = s * XPW

    z16 = jnp.zeros((16,), jnp.float32)
    o16 = jnp.ones((16,), jnp.float32)
    for k in range(CHUNK // 16):
        ones_b[pl.ds(k * 16, 16)] = o16

    def zcnt(i, _):
        cnt_buf[pl.ds(i * 16, 16)] = z16
        return _
    lax.fori_loop(0, CNT_PW // 16, zcnt, None)
    pltpu.sync_copy(cnt_buf, cnts.at[pl.ds(s * CNT_PW, CNT_PW)])

    def zero_rows0():
        def zrow(r, _):
            for k in range(DH // 16):
                rows0[r, pl.ds(k * 16, 16)] = z16
            return _
        lax.fori_loop(0, CHUNK, zrow, None)

    def zero_acc():
        for k in range(APW // CHUNK):
            pltpu.sync_copy(rows0, acc.at[pl.ds(base + k * CHUNK, CHUNK)])

    def stage_x(xh):
        for k in range(5):
            off = xbase + k * 125
            pltpu.sync_copy(xh.at[pl.ds(off, 125)], rows1.at[pl.ds(0, 125)])
            pltpu.sync_copy(rows1.at[pl.ds(0, 125)], xs.at[pl.ds(off, 125)])

    def idesc(r, b):
        return pltpu.make_async_copy(idx_hbm.at[r0 + r], ibuf.at[b], isems[b])

    def gdesc(b, gb):
        rows, sem = gbufs[gb]
        return pltpu.make_async_copy(xs.at[ibuf.at[b, 0]], rows, sem)

    for p, xh in enumerate((x0_hbm, x1_hbm)):
        zero_rows0()
        zero_acc()
        stage_x(xh)
        plsc.subcore_barrier()

        idesc(0, 0).start()
        idesc(1, 1).start()
        idesc(0, 0).wait()
        gdesc(0, 0).start()

        def outer(m, _):
            for b in range(4):
                r = m * 4 + b
                bp1 = (b + 1) % 4
                gb, gbn = b % 2, (b + 1) % 2

                @pl.when(r + 2 < RPW)
                def _pf_idx():
                    idesc(r + 2, (b + 2) % 4).start()

                @pl.when(r + 1 < RPW)
                def _next_gather():
                    idesc(r + 1, bp1).wait()
                    gdesc(bp1, gbn).start()

                gdesc(b, gb).wait()
                rows = gbufs[gb][0]
                didx = ibuf.at[b, 1]
                pltpu.sync_copy(rows, acc.at[didx], add=True)
                if p == 0:
                    pltpu.sync_copy(ones_b, cnts.at[didx], add=True)
            return _
        lax.fori_loop(0, RPW // 4, outer, None)

        plsc.subcore_barrier()

        for k in range(APW // CHUNK):
            off = base + k * CHUNK
            pltpu.sync_copy(acc.at[pl.ds(off, CHUNK)], rows0)
            pltpu.sync_copy(rows0, sum_out.at[p].at[c].at[pl.ds(off, CHUNK)])
        if p == 0:
            pltpu.sync_copy(cnts.at[pl.ds(s * CNT_PW, CNT_PW)], cnt_buf)
            pltpu.sync_copy(cnt_buf, cnt_out.at[c].at[s])
            plsc.subcore_barrier()


@jax.jit
def _aggregate(x0, x1, idx2):
    mesh = plsc.VectorSubcoreMesh(core_axis_name="c", subcore_axis_name="s")
    f = pl.kernel(
        _agg_body,
        out_type=[
            jax.ShapeDtypeStruct((NP, NC, ACC_ROWS, DH), jnp.float32),
            jax.ShapeDtypeStruct((NC, NS, CNT_PW), jnp.float32),
        ],
        mesh=mesh,
        scratch_types=[
            pltpu.VMEM((4, 2, CHUNK), jnp.int32),
            pltpu.VMEM((CHUNK, DH), jnp.float32),
            pltpu.VMEM((CHUNK, DH), jnp.float32),
            pltpu.VMEM((CHUNK,), jnp.float32),
            pltpu.VMEM((CNT_PW,), jnp.float32),
            pltpu.VMEM_SHARED((N, DH), jnp.float32),
            pltpu.VMEM_SHARED((ACC_ROWS, DH), jnp.float32),
            pltpu.VMEM_SHARED((CNT_ROWS,), jnp.float32),
        ] + [pltpu.SemaphoreType.DMA] * 6,
        compiler_params=pltpu.CompilerParams(use_tc_tiling_on_sc=False),
    )
    return f(x0, x1, idx2)


def _tc_body(sum_ref, cnt_ref, x_ref, wlt0_ref, wlt1_ref, wrt_ref, b_ref, o_ref):
    t0 = sum_ref[0, 0] + sum_ref[0, 1]
    t1 = sum_ref[1, 0] + sum_ref[1, 1]
    cnt = cnt_ref[0] + cnt_ref[1]
    rec = 1.0 / jnp.maximum(cnt, 1.0)
    o_ref[...] = (
        jnp.dot(t0 * rec, wlt0_ref[...], preferred_element_type=jnp.float32)
        + jnp.dot(t1 * rec, wlt1_ref[...], preferred_element_type=jnp.float32)
        + jnp.dot(x_ref[...], wrt_ref[...], preferred_element_type=jnp.float32)
        + b_ref[...]
    )


@jax.jit
def _combine(summed, cnt, x, wlt0, wlt1, wrt, b):
    blk = 1000
    grid = N // blk
    return pl.pallas_call(
        _tc_body,
        grid=(grid,),
        in_specs=[
            pl.BlockSpec((NP, NC, blk, DH), lambda i: (0, 0, i, 0)),
            pl.BlockSpec((NC, blk, 1), lambda i: (0, i, 0)),
            pl.BlockSpec((blk, D), lambda i: (i, 0)),
            pl.BlockSpec((DH, D), lambda i: (0, 0)),
            pl.BlockSpec((DH, D), lambda i: (0, 0)),
            pl.BlockSpec((D, D), lambda i: (0, 0)),
            pl.BlockSpec((1, D), lambda i: (0, 0)),
        ],
        out_specs=pl.BlockSpec((blk, D), lambda i: (i, 0)),
        out_shape=jax.ShapeDtypeStruct((N, D), jnp.float32),
    )(summed, cnt, x, wlt0, wlt1, wrt, b)


def kernel(x, edge_index, W_l, b_l, W_r):
    src = edge_index[0].astype(jnp.int32)
    dst = edge_index[1].astype(jnp.int32)
    pad = EP - E
    src2 = jnp.concatenate([src, jnp.zeros((pad,), jnp.int32)]).reshape(ROWS, CHUNK)
    dst2 = jnp.concatenate([dst, jnp.full((pad,), N, jnp.int32)]).reshape(ROWS, CHUNK)
    idx2 = jnp.stack([src2, dst2], axis=1)
    wlt = W_l.T
    summed, cnts = _aggregate(x[:, :DH], x[:, DH:], idx2)
    cnt = cnts.reshape(NC, CNT_ROWS, 1)[:, :N]
    return _combine(summed, cnt, x, wlt[:DH], wlt[DH:], W_r.T, b_l.reshape(1, D))


# speedup vs baseline: 2.1882x; 1.0531x over previous
import functools

import jax
import jax.numpy as jnp
from jax import lax
from jax.experimental import pallas as pl
from jax.experimental.pallas import tpu as pltpu
from jax.experimental.pallas import tpu_sc as plsc

N = 10000
D = 128
E = 320000
NC = 2
NS = 16
NW = NC * NS
DH = 64
NP = 2
CHUNK = 128
RPW = 80
EP = NW * RPW * CHUNK
ROWS = EP // CHUNK
ACC_ROWS = 10240
CNT_ROWS = 10240
CNT_PW = CNT_ROWS // NS
APW = ACC_ROWS // NS
XPW = N // NS


def _agg_body(x0_hbm, x1_hbm, idx_hbm, sum_out, cnt_out,
              ibuf, rows0, rows1, ones_b, cnt_buf,
              xs, acc, cnts,
              isem0, isem1, isem2, isem3, gsem0, gsem1, csem0, csem1):
    c = lax.axis_index("c")
    s = lax.axis_index("s")
    wid = s * NC + c
    r0 = wid * RPW
    isems = (isem0, isem1, isem2, isem3)
    gbufs = ((rows0, gsem0), (rows1, gsem1))
    csems = (csem0, csem1)
    base = s * APW
    xbase = s * XPW

    z16 = jnp.zeros((16,), jnp.float32)
    o16 = jnp.ones((16,), jnp.float32)
    for k in range(CHUNK // 16):
        ones_b[pl.ds(k * 16, 16)] = o16

    def zcnt(i, _):
        cnt_buf[pl.ds(i * 16, 16)] = z16
        return _
    lax.fori_loop(0, CNT_PW // 16, zcnt, None)
    pltpu.sync_copy(cnt_buf, cnts.at[pl.ds(s * CNT_PW, CNT_PW)])

    def zero_rows0():
        def zrow(r, _):
            for k in range(DH // 16):
                rows0[r, pl.ds(k * 16, 16)] = z16
            return _
        lax.fori_loop(0, CHUNK, zrow, None)

    def zero_acc():
        for k in range(APW // CHUNK):
            pltpu.sync_copy(rows0, acc.at[pl.ds(base + k * CHUNK, CHUNK)])

    def stage_x(xh):
        for k in range(5):
            off = xbase + k * 125
            pltpu.sync_copy(xh.at[pl.ds(off, 125)], rows1.at[pl.ds(0, 125)])
            pltpu.sync_copy(rows1.at[pl.ds(0, 125)], xs.at[pl.ds(off, 125)])

    def idesc(r, b):
        return pltpu.make_async_copy(idx_hbm.at[r0 + r], ibuf.at[b], isems[b])

    def gdesc(b, gb):
        rows, sem = gbufs[gb]
        return pltpu.make_async_copy(xs.at[ibuf.at[b, 0]], rows, sem)

    def cdesc(b, gb):
        return pltpu.make_async_copy(ones_b, cnts.at[ibuf.at[b, 1]], csems[gb])

    for p, xh in enumerate((x0_hbm, x1_hbm)):
        zero_rows0()
        zero_acc()
        stage_x(xh)
        plsc.subcore_barrier()

        idesc(0, 0).start()
        idesc(1, 1).start()
        idesc(0, 0).wait()
        gdesc(0, 0).start()

        def outer(m, _):
            for b in range(4):
                r = m * 4 + b
                bp1 = (b + 1) % 4
                gb, gbn = b % 2, (b + 1) % 2

                @pl.when(r + 2 < RPW)
                def _pf_idx():
                    idesc(r + 2, (b + 2) % 4).start()

                @pl.when(r + 1 < RPW)
                def _next_gather():
                    idesc(r + 1, bp1).wait()
                    gdesc(bp1, gbn).start()

                gdesc(b, gb).wait()
                if p == 0:
                    @pl.when(r >= 1)
                    def _drain_cnt():
                        cdesc((b - 1) % 4, gbn).wait()
                rows = gbufs[gb][0]
                didx = ibuf.at[b, 1]
                pltpu.sync_copy(rows, acc.at[didx], add=True)
                if p == 0:
                    pltpu.async_copy(ones_b, cnts.at[didx], csems[gb], add=True)
            return _
        lax.fori_loop(0, RPW // 4, outer, None)
        if p == 0:
            cdesc(3, 1).wait()

        plsc.subcore_barrier()

        for k in range(APW // CHUNK):
            off = base + k * CHUNK
            pltpu.sync_copy(acc.at[pl.ds(off, CHUNK)], rows0)
            pltpu.sync_copy(rows0, sum_out.at[p].at[c].at[pl.ds(off, CHUNK)])
        if p == 0:
            pltpu.sync_copy(cnts.at[pl.ds(s * CNT_PW, CNT_PW)], cnt_buf)
            pltpu.sync_copy(cnt_buf, cnt_out.at[c].at[pl.ds(s * CNT_PW, CNT_PW)])
            plsc.subcore_barrier()


@jax.jit
def _aggregate(x0, x1, idx2):
    mesh = plsc.VectorSubcoreMesh(core_axis_name="c", subcore_axis_name="s")
    f = pl.kernel(
        _agg_body,
        out_type=[
            jax.ShapeDtypeStruct((NP, NC, ACC_ROWS, DH), jnp.float32),
            jax.ShapeDtypeStruct((NC, CNT_ROWS), jnp.float32),
        ],
        mesh=mesh,
        scratch_types=[
            pltpu.VMEM((4, 2, CHUNK), jnp.int32),
            pltpu.VMEM((CHUNK, DH), jnp.float32),
            pltpu.VMEM((CHUNK, DH), jnp.float32),
            pltpu.VMEM((CHUNK,), jnp.float32),
            pltpu.VMEM((CNT_PW,), jnp.float32),
            pltpu.VMEM_SHARED((N, DH), jnp.float32),
            pltpu.VMEM_SHARED((ACC_ROWS, DH), jnp.float32),
            pltpu.VMEM_SHARED((CNT_ROWS,), jnp.float32),
        ] + [pltpu.SemaphoreType.DMA] * 8,
        compiler_params=pltpu.CompilerParams(use_tc_tiling_on_sc=False),
    )
    return f(x0, x1, idx2)


def _tc_body(sum_ref, cnt_ref, x_ref, wlt0_ref, wlt1_ref, wrt_ref, b_ref, o_ref):
    t0 = sum_ref[0, 0] + sum_ref[0, 1]
    t1 = sum_ref[1, 0] + sum_ref[1, 1]
    cnt = cnt_ref[0] + cnt_ref[1]
    rec = 1.0 / jnp.maximum(cnt, 1.0)
    o_ref[...] = (
        jnp.dot(t0 * rec, wlt0_ref[...], preferred_element_type=jnp.float32)
        + jnp.dot(t1 * rec, wlt1_ref[...], preferred_element_type=jnp.float32)
        + jnp.dot(x_ref[...], wrt_ref[...], preferred_element_type=jnp.float32)
        + b_ref[...]
    )


@jax.jit
def _combine(summed, cnt, x, wlt0, wlt1, wrt, b):
    blk = 1000
    grid = N // blk
    return pl.pallas_call(
        _tc_body,
        grid=(grid,),
        in_specs=[
            pl.BlockSpec((NP, NC, blk, DH), lambda i: (0, 0, i, 0)),
            pl.BlockSpec((NC, blk, 1), lambda i: (0, i, 0)),
            pl.BlockSpec((blk, D), lambda i: (i, 0)),
            pl.BlockSpec((DH, D), lambda i: (0, 0)),
            pl.BlockSpec((DH, D), lambda i: (0, 0)),
            pl.BlockSpec((D, D), lambda i: (0, 0)),
            pl.BlockSpec((1, D), lambda i: (0, 0)),
        ],
        out_specs=pl.BlockSpec((blk, D), lambda i: (i, 0)),
        out_shape=jax.ShapeDtypeStruct((N, D), jnp.float32),
    )(summed, cnt, x, wlt0, wlt1, wrt, b)


def kernel(x, edge_index, W_l, b_l, W_r):
    src = edge_index[0].astype(jnp.int32)
    dst = edge_index[1].astype(jnp.int32)
    pad = EP - E
    src2 = jnp.concatenate([src, jnp.zeros((pad,), jnp.int32)]).reshape(ROWS, CHUNK)
    dst2 = jnp.concatenate([dst, jnp.full((pad,), N, jnp.int32)]).reshape(ROWS, CHUNK)
    idx2 = jnp.stack([src2, dst2], axis=1)
    wlt = W_l.T
    summed, cnts = _aggregate(x[:, :DH], x[:, DH:], idx2)
    cnt = cnts[:, :N, None]
    return _combine(summed, cnt, x, wlt[:DH], wlt[DH:], W_r.T, b_l.reshape(1, D))
